# Initial kernel scaffold; baseline (speedup 1.0000x reference)
import jax
import jax.numpy as jnp
from jax.experimental import pallas as pl


def kernel(node_feats, edge_index, num_graphs, W1, b1, W2, b2, W3, b3):
    raise NotImplementedError("write your pallas kernel here")


# speedup vs baseline: 6.4099x; 6.4099x over previous
import functools

import jax
import jax.numpy as jnp
from jax import lax
from jax.experimental import pallas as pl
from jax.experimental.pallas import tpu as pltpu
from jax.experimental.pallas import tpu_sc as plsc

N = 10000
D = 128
E = 320000
NC = 2
NS = 16
NW = NC * NS
CH = 80
EPT = E // NW
NCHUNK = EPT // CH
WB_TILES = 10
WROWS = N // WB_TILES
BROWS = 40
R = 1000


def _mesh():
    return plsc.VectorSubcoreMesh(core_axis_name="c", subcore_axis_name="s")


@functools.partial(
    pl.kernel,
    out_type=(
        jax.ShapeDtypeStruct((NC * N,), jnp.float32),
        jax.ShapeDtypeStruct((NC * N,), jnp.float32),
    ),
    mesh=_mesh(),
    scratch_types=[
        pltpu.VMEM((NCHUNK, CH), jnp.int32),
        pltpu.VMEM((NCHUNK, CH), jnp.int32),
        pltpu.VMEM((CH,), jnp.float32),
        pltpu.VMEM((WROWS,), jnp.float32),
        pltpu.VMEM_SHARED((N,), jnp.float32),
        pltpu.VMEM_SHARED((N,), jnp.float32),
    ],
)
def _degree_kernel(src_hbm, dst_hbm, din_hbm, dout_hbm, sidx, didx, ones, zbuf,
                   acc_in, acc_out):
    c = lax.axis_index("c")
    s = lax.axis_index("s")
    w = c * NS + s
    for j in range(CH // 16):
        ones[pl.ds(j * 16, 16)] = jnp.ones((16,), jnp.float32)

    def _z(i, carry):
        zbuf[pl.ds(i * 16, 16)] = jnp.zeros((16,), jnp.float32)
        return carry

    lax.fori_loop(0, WROWS // 16, _z, 0)

    @pl.when(s < WB_TILES)
    def _():
        pltpu.sync_copy(zbuf, acc_in.at[pl.ds(s * WROWS, WROWS)])
        pltpu.sync_copy(zbuf, acc_out.at[pl.ds(s * WROWS, WROWS)])

    plsc.subcore_barrier()
    pltpu.sync_copy(src_hbm.at[w], sidx)
    pltpu.sync_copy(dst_hbm.at[w], didx)

    def _body(j, carry):
        pltpu.sync_copy(ones, acc_in.at[didx.at[j]], add=True)
        pltpu.sync_copy(ones, acc_out.at[sidx.at[j]], add=True)
        return carry

    lax.fori_loop(0, NCHUNK, _body, 0)
    plsc.subcore_barrier()

    @pl.when(s < WB_TILES)
    def _():
        pltpu.sync_copy(acc_in.at[pl.ds(s * WROWS, WROWS)], zbuf)
        pltpu.sync_copy(zbuf, din_hbm.at[pl.ds(c * N + s * WROWS, WROWS)])
        pltpu.sync_copy(acc_out.at[pl.ds(s * WROWS, WROWS)], zbuf)
        pltpu.sync_copy(zbuf, dout_hbm.at[pl.ds(c * N + s * WROWS, WROWS)])


@functools.partial(
    pl.kernel,
    out_type=jax.ShapeDtypeStruct((NC, N, D), jnp.float32),
    mesh=_mesh(),
    scratch_types=[
        pltpu.VMEM((NCHUNK, CH), jnp.int32),
        pltpu.VMEM((NCHUNK, CH), jnp.int32),
        pltpu.VMEM((CH, D), jnp.float32),
        pltpu.VMEM((BROWS, D), jnp.float32),
        pltpu.VMEM_SHARED((N, D), jnp.float32),
        pltpu.SemaphoreType.DMA,
    ],
)
def _agg_kernel(u_hbm, src_hbm, dst_hbm, out_hbm, sidx, didx, rows, zbuf, acc, sem):
    c = lax.axis_index("c")
    s = lax.axis_index("s")
    w = c * NS + s

    def _z(i, carry):
        for j in range(D // 16):
            zbuf[i, pl.ds(j * 16, 16)] = jnp.zeros((16,), jnp.float32)
        return carry

    lax.fori_loop(0, BROWS, _z, 0)

    @pl.when(s < WB_TILES)
    def _():
        for k in range(WROWS // BROWS):
            pltpu.sync_copy(zbuf, acc.at[pl.ds(s * WROWS + k * BROWS, BROWS)])

    plsc.subcore_barrier()

    pltpu.sync_copy(src_hbm.at[w], sidx)
    pltpu.sync_copy(dst_hbm.at[w], didx)

    def _body(j, carry):
        pltpu.async_copy(u_hbm.at[sidx.at[j]], rows, sem).wait()
        pltpu.sync_copy(rows, acc.at[didx.at[j]], add=True)
        return carry

    lax.fori_loop(0, NCHUNK, _body, 0)
    plsc.subcore_barrier()

    @pl.when(s < WB_TILES)
    def _():
        for k in range(WROWS // BROWS):
            r0 = s * WROWS + k * BROWS
            pltpu.sync_copy(acc.at[pl.ds(r0, BROWS)], zbuf)
            pltpu.sync_copy(zbuf, out_hbm.at[c, pl.ds(r0, BROWS)])


def _prep_body(dip_ref, dop_ref, x_ref, u0_ref, nin_ref, nout_ref):
    d_in = dip_ref[0] + dip_ref[1]
    d_out = dop_ref[0] + dop_ref[1]
    nin = jnp.where(d_in > 0, lax.rsqrt(jnp.maximum(d_in, 1.0)), 0.0)
    nout = jnp.where(d_out > 0, lax.rsqrt(jnp.maximum(d_out, 1.0)), 0.0)
    nin_ref[...] = nin
    nout_ref[...] = nout
    u0_ref[...] = x_ref[...] * nout


_prep = pl.pallas_call(
    _prep_body,
    grid=(N // R,),
    in_specs=[
        pl.BlockSpec((NC, R, 1), lambda i: (0, i, 0)),
        pl.BlockSpec((NC, R, 1), lambda i: (0, i, 0)),
        pl.BlockSpec((R, D), lambda i: (i, 0)),
    ],
    out_specs=[
        pl.BlockSpec((R, D), lambda i: (i, 0)),
        pl.BlockSpec((R, 1), lambda i: (i, 0)),
        pl.BlockSpec((R, 1), lambda i: (i, 0)),
    ],
    out_shape=[
        jax.ShapeDtypeStruct((N, D), jnp.float32),
        jax.ShapeDtypeStruct((N, 1), jnp.float32),
        jax.ShapeDtypeStruct((N, 1), jnp.float32),
    ],
)


def _make_layer(relu, scale_out):
    def body(aggp_ref, nin_ref, nout_ref, w_ref, b_ref, c_ref, o_ref):
        h = (aggp_ref[0] + aggp_ref[1]) * nin_ref[...]
        z = jnp.dot(h, w_ref[...], preferred_element_type=jnp.float32)
        z = z + b_ref[...] + c_ref[...]
        if relu:
            z = jnp.maximum(z, 0.0)
        if scale_out:
            z = z * nout_ref[...]
        o_ref[...] = z

    return pl.pallas_call(
        body,
        grid=(N // R,),
        in_specs=[
            pl.BlockSpec((NC, R, D), lambda i: (0, i, 0)),
            pl.BlockSpec((R, 1), lambda i: (i, 0)),
            pl.BlockSpec((R, 1), lambda i: (i, 0)),
            pl.BlockSpec((D, D), lambda i: (0, 0)),
            pl.BlockSpec((1, D), lambda i: (0, 0)),
            pl.BlockSpec((1, 1), lambda i: (0, 0)),
        ],
        out_specs=pl.BlockSpec((R, D), lambda i: (i, 0)),
        out_shape=jax.ShapeDtypeStruct((N, D), jnp.float32),
    )


_layer_mid = _make_layer(relu=True, scale_out=True)
_layer_last = _make_layer(relu=False, scale_out=False)


def kernel(node_feats, edge_index, num_graphs, W1, b1, W2, b2, W3, b3):
    ei = edge_index.astype(jnp.int32)
    src2d = ei[0].reshape(NW, NCHUNK, CH)
    dst2d = ei[1].reshape(NW, NCHUNK, CH)

    din_p, dout_p = _degree_kernel(src2d, dst2d)
    u, nin, nout = _prep(din_p.reshape(NC, N, 1), dout_p.reshape(NC, N, 1),
                         node_feats)

    zero_c = jnp.zeros((1, 1), jnp.float32)
    cadd = (jnp.asarray(num_graphs) - 8).astype(jnp.float32).reshape(1, 1)

    aggp = _agg_kernel(u, src2d, dst2d)
    u = _layer_mid(aggp, nin, nout, W1, b1.reshape(1, D), zero_c)
    aggp = _agg_kernel(u, src2d, dst2d)
    u = _layer_mid(aggp, nin, nout, W2, b2.reshape(1, D), zero_c)
    aggp = _agg_kernel(u, src2d, dst2d)
    h = _layer_last(aggp, nin, nout, W3, b3.reshape(1, D), cadd)

    return h.reshape(8, N // 8, D)


# speedup vs baseline: 7.7461x; 1.2085x over previous
import functools

import jax
import jax.numpy as jnp
from jax import lax
from jax.experimental import pallas as pl
from jax.experimental.pallas import tpu as pltpu
from jax.experimental.pallas import tpu_sc as plsc

N = 10000
D = 128
E = 320000
NC = 2
NS = 16
NW = NC * NS
CH = 80
EPT = E // NW
NCHUNK = EPT // CH
SUP = 25
NSEG = NCHUNK // SUP
WB_TILES = 10
WROWS = N // WB_TILES
BROWS = 200
R = 1000


def _mesh():
    return plsc.VectorSubcoreMesh(core_axis_name="c", subcore_axis_name="s")


@functools.partial(
    pl.kernel,
    out_type=(
        jax.ShapeDtypeStruct((NC * N,), jnp.float32),
        jax.ShapeDtypeStruct((NC * N,), jnp.float32),
    ),
    mesh=_mesh(),
    scratch_types=[
        pltpu.VMEM((NCHUNK, CH), jnp.int32),
        pltpu.VMEM((NCHUNK, CH), jnp.int32),
        pltpu.VMEM((CH,), jnp.float32),
        pltpu.VMEM((WROWS,), jnp.float32),
        pltpu.VMEM_SHARED((N,), jnp.float32),
        pltpu.VMEM_SHARED((N,), jnp.float32),
    ],
)
def _degree_kernel(src_hbm, dst_hbm, din_hbm, dout_hbm, sidx, didx, ones, zbuf,
                   acc_in, acc_out):
    c = lax.axis_index("c")
    s = lax.axis_index("s")
    w = c * NS + s
    for j in range(CH // 16):
        ones[pl.ds(j * 16, 16)] = jnp.ones((16,), jnp.float32)

    def _z(i, carry):
        zbuf[pl.ds(i * 16, 16)] = jnp.zeros((16,), jnp.float32)
        return carry

    lax.fori_loop(0, WROWS // 16, _z, 0)

    @pl.when(s < WB_TILES)
    def _():
        pltpu.sync_copy(zbuf, acc_in.at[pl.ds(s * WROWS, WROWS)])
        pltpu.sync_copy(zbuf, acc_out.at[pl.ds(s * WROWS, WROWS)])

    plsc.subcore_barrier()
    pltpu.sync_copy(src_hbm.at[w], sidx)
    pltpu.sync_copy(dst_hbm.at[w], didx)

    def _body(j, carry):
        pltpu.sync_copy(ones, acc_in.at[didx.at[j]], add=True)
        pltpu.sync_copy(ones, acc_out.at[sidx.at[j]], add=True)
        return carry

    lax.fori_loop(0, NCHUNK, _body, 0)
    plsc.subcore_barrier()

    @pl.when(s < WB_TILES)
    def _():
        pltpu.sync_copy(acc_in.at[pl.ds(s * WROWS, WROWS)], zbuf)
        pltpu.sync_copy(zbuf, din_hbm.at[pl.ds(c * N + s * WROWS, WROWS)])
        pltpu.sync_copy(acc_out.at[pl.ds(s * WROWS, WROWS)], zbuf)
        pltpu.sync_copy(zbuf, dout_hbm.at[pl.ds(c * N + s * WROWS, WROWS)])


@functools.partial(
    pl.kernel,
    out_type=jax.ShapeDtypeStruct((NC, N, D), jnp.float32),
    mesh=_mesh(),
    scratch_types=[
        pltpu.VMEM((SUP, CH), jnp.int32),
        pltpu.VMEM((SUP, CH), jnp.int32),
        pltpu.VMEM((CH, D), jnp.float32),
        pltpu.VMEM((CH, D), jnp.float32),
        pltpu.VMEM_SHARED((N, D), jnp.float32),
        pltpu.SemaphoreType.DMA,
        pltpu.SemaphoreType.DMA,
    ],
)
def _agg_kernel(u_hbm, src_hbm, dst_hbm, zero_hbm, out_hbm, sidx, didx,
                rows_a, rows_b, acc, sem_a, sem_b):
    c = lax.axis_index("c")
    s = lax.axis_index("s")
    w = c * NS + s

    @pl.when(s < WB_TILES)
    def _():
        for k in range(WROWS // BROWS):
            pltpu.sync_copy(zero_hbm, acc.at[pl.ds(s * WROWS + k * BROWS, BROWS)])

    plsc.subcore_barrier()

    def _fire(j, rows, sem):
        return pltpu.async_copy(u_hbm.at[sidx.at[j]], rows, sem)

    def _scatter(j, rows):
        pltpu.sync_copy(rows, acc.at[didx.at[j]], add=True)

    slots = ((rows_a, sem_a), (rows_b, sem_b))

    def _sup(g, carry):
        pltpu.sync_copy(src_hbm.at[w, g], sidx)
        pltpu.sync_copy(dst_hbm.at[w, g], didx)
        _fire(0, *slots[0]).wait()
        for t in range(SUP):
            nxt = None
            if t + 1 < SUP:
                nxt = _fire(t + 1, *slots[(t + 1) % 2])
            _scatter(t, slots[t % 2][0])
            if nxt is not None:
                nxt.wait()
        return carry

    lax.fori_loop(0, NSEG, _sup, 0)
    plsc.subcore_barrier()

    @pl.when(s < WB_TILES)
    def _():
        for k in range(WROWS // BROWS):
            r0 = s * WROWS + k * BROWS
            pltpu.sync_copy(acc.at[pl.ds(r0, BROWS)], out_hbm.at[c, pl.ds(r0, BROWS)])


def _prep_body(dip_ref, dop_ref, x_ref, u0_ref, nin_ref, nout_ref):
    d_in = dip_ref[0] + dip_ref[1]
    d_out = dop_ref[0] + dop_ref[1]
    nin = jnp.where(d_in > 0, lax.rsqrt(jnp.maximum(d_in, 1.0)), 0.0)
    nout = jnp.where(d_out > 0, lax.rsqrt(jnp.maximum(d_out, 1.0)), 0.0)
    nin_ref[...] = nin
    nout_ref[...] = nout
    u0_ref[...] = x_ref[...] * nout


_prep = pl.pallas_call(
    _prep_body,
    grid=(N // R,),
    in_specs=[
        pl.BlockSpec((NC, R, 1), lambda i: (0, i, 0)),
        pl.BlockSpec((NC, R, 1), lambda i: (0, i, 0)),
        pl.BlockSpec((R, D), lambda i: (i, 0)),
    ],
    out_specs=[
        pl.BlockSpec((R, D), lambda i: (i, 0)),
        pl.BlockSpec((R, 1), lambda i: (i, 0)),
        pl.BlockSpec((R, 1), lambda i: (i, 0)),
    ],
    out_shape=[
        jax.ShapeDtypeStruct((N, D), jnp.float32),
        jax.ShapeDtypeStruct((N, 1), jnp.float32),
        jax.ShapeDtypeStruct((N, 1), jnp.float32),
    ],
)


def _make_layer(relu, scale_out):
    def body(aggp_ref, nin_ref, nout_ref, w_ref, b_ref, c_ref, o_ref):
        h = (aggp_ref[0] + aggp_ref[1]) * nin_ref[...]
        z = jnp.dot(h, w_ref[...], preferred_element_type=jnp.float32)
        z = z + b_ref[...] + c_ref[...]
        if relu:
            z = jnp.maximum(z, 0.0)
        if scale_out:
            z = z * nout_ref[...]
        o_ref[...] = z

    return pl.pallas_call(
        body,
        grid=(N // R,),
        in_specs=[
            pl.BlockSpec((NC, R, D), lambda i: (0, i, 0)),
            pl.BlockSpec((R, 1), lambda i: (i, 0)),
            pl.BlockSpec((R, 1), lambda i: (i, 0)),
            pl.BlockSpec((D, D), lambda i: (0, 0)),
            pl.BlockSpec((1, D), lambda i: (0, 0)),
            pl.BlockSpec((1, 1), lambda i: (0, 0)),
        ],
        out_specs=pl.BlockSpec((R, D), lambda i: (i, 0)),
        out_shape=jax.ShapeDtypeStruct((N, D), jnp.float32),
    )


_layer_mid = _make_layer(relu=True, scale_out=True)
_layer_last = _make_layer(relu=False, scale_out=False)


def kernel(node_feats, edge_index, num_graphs, W1, b1, W2, b2, W3, b3):
    ei = edge_index.astype(jnp.int32)
    src3d = ei[0].reshape(NW, NCHUNK, CH)
    dst3d = ei[1].reshape(NW, NCHUNK, CH)
    src4d = src3d.reshape(NW, NSEG, SUP, CH)
    dst4d = dst3d.reshape(NW, NSEG, SUP, CH)
    zero_rows = jnp.zeros((BROWS, D), jnp.float32)

    din_p, dout_p = _degree_kernel(src3d, dst3d)
    u, nin, nout = _prep(din_p.reshape(NC, N, 1), dout_p.reshape(NC, N, 1),
                         node_feats)

    zero_c = jnp.zeros((1, 1), jnp.float32)
    cadd = (jnp.asarray(num_graphs) - 8).astype(jnp.float32).reshape(1, 1)

    aggp = _agg_kernel(u, src4d, dst4d, zero_rows)
    u = _layer_mid(aggp, nin, nout, W1, b1.reshape(1, D), zero_c)
    aggp = _agg_kernel(u, src4d, dst4d, zero_rows)
    u = _layer_mid(aggp, nin, nout, W2, b2.reshape(1, D), zero_c)
    aggp = _agg_kernel(u, src4d, dst4d, zero_rows)
    h = _layer_last(aggp, nin, nout, W3, b3.reshape(1, D), cadd)

    return h.reshape(8, N // 8, D)


# speedup vs baseline: 7.8909x; 1.0187x over previous
import functools

import jax
import jax.numpy as jnp
from jax import lax
from jax.experimental import pallas as pl
from jax.experimental.pallas import tpu as pltpu
from jax.experimental.pallas import tpu_sc as plsc

N = 10000
D = 128
E = 320000
NC = 2
NS = 16
NW = NC * NS
CH = 80
EPT = E // NW
NCHUNK = EPT // CH
SUP = 25
NSEG = NCHUNK // SUP
WB_TILES = 10
WROWS = N // WB_TILES
BROWS = 1000
R = 1000


def _mesh():
    return plsc.VectorSubcoreMesh(core_axis_name="c", subcore_axis_name="s")


@functools.partial(
    pl.kernel,
    out_type=(
        jax.ShapeDtypeStruct((NC * N,), jnp.float32),
        jax.ShapeDtypeStruct((NC * N,), jnp.float32),
    ),
    mesh=_mesh(),
    scratch_types=[
        pltpu.VMEM((NCHUNK, CH), jnp.int32),
        pltpu.VMEM((NCHUNK, CH), jnp.int32),
        pltpu.VMEM((CH,), jnp.float32),
        pltpu.VMEM((WROWS,), jnp.float32),
        pltpu.VMEM_SHARED((N,), jnp.float32),
        pltpu.VMEM_SHARED((N,), jnp.float32),
    ],
)
def _degree_kernel(src_hbm, dst_hbm, din_hbm, dout_hbm, sidx, didx, ones, zbuf,
                   acc_in, acc_out):
    c = lax.axis_index("c")
    s = lax.axis_index("s")
    w = c * NS + s
    for j in range(CH // 16):
        ones[pl.ds(j * 16, 16)] = jnp.ones((16,), jnp.float32)

    def _z(i, carry):
        zbuf[pl.ds(i * 16, 16)] = jnp.zeros((16,), jnp.float32)
        return carry

    lax.fori_loop(0, WROWS // 16, _z, 0)

    @pl.when(s < WB_TILES)
    def _():
        pltpu.sync_copy(zbuf, acc_in.at[pl.ds(s * WROWS, WROWS)])
        pltpu.sync_copy(zbuf, acc_out.at[pl.ds(s * WROWS, WROWS)])

    plsc.subcore_barrier()
    pltpu.sync_copy(src_hbm.at[w], sidx)
    pltpu.sync_copy(dst_hbm.at[w], didx)

    def _body(j, carry):
        pltpu.sync_copy(ones, acc_in.at[didx.at[j]], add=True)
        pltpu.sync_copy(ones, acc_out.at[sidx.at[j]], add=True)
        return carry

    lax.fori_loop(0, NCHUNK, _body, 0)
    plsc.subcore_barrier()

    @pl.when(s < WB_TILES)
    def _():
        pltpu.sync_copy(acc_in.at[pl.ds(s * WROWS, WROWS)], zbuf)
        pltpu.sync_copy(zbuf, din_hbm.at[pl.ds(c * N + s * WROWS, WROWS)])
        pltpu.sync_copy(acc_out.at[pl.ds(s * WROWS, WROWS)], zbuf)
        pltpu.sync_copy(zbuf, dout_hbm.at[pl.ds(c * N + s * WROWS, WROWS)])


@functools.partial(
    pl.kernel,
    out_type=jax.ShapeDtypeStruct((NC, N, D), jnp.float32),
    mesh=_mesh(),
    scratch_types=[
        pltpu.VMEM((SUP, CH), jnp.int32),
        pltpu.VMEM((SUP, CH), jnp.int32),
        pltpu.VMEM((CH, D), jnp.float32),
        pltpu.VMEM((CH, D), jnp.float32),
        pltpu.VMEM_SHARED((N, D), jnp.float32),
        pltpu.SemaphoreType.DMA,
        pltpu.SemaphoreType.DMA,
    ],
)
def _agg_kernel(u_hbm, src_hbm, dst_hbm, zero_hbm, out_hbm, sidx, didx,
                rows_a, rows_b, acc, sem_a, sem_b):
    c = lax.axis_index("c")
    s = lax.axis_index("s")
    w = c * NS + s

    @pl.when(s < WB_TILES)
    def _():
        for k in range(WROWS // BROWS):
            pltpu.sync_copy(zero_hbm, acc.at[pl.ds(s * WROWS + k * BROWS, BROWS)])

    plsc.subcore_barrier()

    def _fire(j, rows, sem):
        return pltpu.async_copy(u_hbm.at[sidx.at[j]], rows, sem)

    def _scatter(j, rows):
        pltpu.sync_copy(rows, acc.at[didx.at[j]], add=True)

    slots = ((rows_a, sem_a), (rows_b, sem_b))

    def _sup(g, carry):
        pltpu.sync_copy(src_hbm.at[w, g], sidx)
        pltpu.sync_copy(dst_hbm.at[w, g], didx)
        _fire(0, *slots[0]).wait()
        for t in range(SUP):
            nxt = None
            if t + 1 < SUP:
                nxt = _fire(t + 1, *slots[(t + 1) % 2])
            _scatter(t, slots[t % 2][0])
            if nxt is not None:
                nxt.wait()
        return carry

    lax.fori_loop(0, NSEG, _sup, 0)
    plsc.subcore_barrier()

    @pl.when(s < WB_TILES)
    def _():
        for k in range(WROWS // BROWS):
            r0 = s * WROWS + k * BROWS
            pltpu.sync_copy(acc.at[pl.ds(r0, BROWS)], out_hbm.at[c, pl.ds(r0, BROWS)])


def _prep_body(dip_ref, dop_ref, x_ref, u0_ref, nin_ref, nout_ref):
    d_in = dip_ref[0] + dip_ref[1]
    d_out = dop_ref[0] + dop_ref[1]
    nin = jnp.where(d_in > 0, lax.rsqrt(jnp.maximum(d_in, 1.0)), 0.0)
    nout = jnp.where(d_out > 0, lax.rsqrt(jnp.maximum(d_out, 1.0)), 0.0)
    nin_ref[...] = nin
    nout_ref[...] = nout
    u0_ref[...] = x_ref[...] * nout


_prep = pl.pallas_call(
    _prep_body,
    grid=(N // R,),
    in_specs=[
        pl.BlockSpec((NC, R, 1), lambda i: (0, i, 0)),
        pl.BlockSpec((NC, R, 1), lambda i: (0, i, 0)),
        pl.BlockSpec((R, D), lambda i: (i, 0)),
    ],
    out_specs=[
        pl.BlockSpec((R, D), lambda i: (i, 0)),
        pl.BlockSpec((R, 1), lambda i: (i, 0)),
        pl.BlockSpec((R, 1), lambda i: (i, 0)),
    ],
    out_shape=[
        jax.ShapeDtypeStruct((N, D), jnp.float32),
        jax.ShapeDtypeStruct((N, 1), jnp.float32),
        jax.ShapeDtypeStruct((N, 1), jnp.float32),
    ],
)


def _make_layer(relu, scale_out):
    def body(aggp_ref, nin_ref, nout_ref, w_ref, b_ref, c_ref, o_ref):
        h = (aggp_ref[0] + aggp_ref[1]) * nin_ref[...]
        z = jnp.dot(h, w_ref[...], preferred_element_type=jnp.float32)
        z = z + b_ref[...] + c_ref[...]
        if relu:
            z = jnp.maximum(z, 0.0)
        if scale_out:
            z = z * nout_ref[...]
        o_ref[...] = z

    return pl.pallas_call(
        body,
        grid=(N // R,),
        in_specs=[
            pl.BlockSpec((NC, R, D), lambda i: (0, i, 0)),
            pl.BlockSpec((R, 1), lambda i: (i, 0)),
            pl.BlockSpec((R, 1), lambda i: (i, 0)),
            pl.BlockSpec((D, D), lambda i: (0, 0)),
            pl.BlockSpec((1, D), lambda i: (0, 0)),
            pl.BlockSpec((1, 1), lambda i: (0, 0)),
        ],
        out_specs=pl.BlockSpec((R, D), lambda i: (i, 0)),
        out_shape=jax.ShapeDtypeStruct((N, D), jnp.float32),
    )


_layer_mid = _make_layer(relu=True, scale_out=True)
_layer_last = _make_layer(relu=False, scale_out=False)


def kernel(node_feats, edge_index, num_graphs, W1, b1, W2, b2, W3, b3):
    ei = edge_index.astype(jnp.int32)
    src3d = ei[0].reshape(NW, NCHUNK, CH)
    dst3d = ei[1].reshape(NW, NCHUNK, CH)
    src4d = src3d.reshape(NW, NSEG, SUP, CH)
    dst4d = dst3d.reshape(NW, NSEG, SUP, CH)
    zero_rows = jnp.zeros((BROWS, D), jnp.float32)

    din_p, dout_p = _degree_kernel(src3d, dst3d)
    u, nin, nout = _prep(din_p.reshape(NC, N, 1), dout_p.reshape(NC, N, 1),
                         node_feats)

    zero_c = jnp.zeros((1, 1), jnp.float32)
    cadd = (jnp.asarray(num_graphs) - 8).astype(jnp.float32).reshape(1, 1)

    aggp = _agg_kernel(u, src4d, dst4d, zero_rows)
    u = _layer_mid(aggp, nin, nout, W1, b1.reshape(1, D), zero_c)
    aggp = _agg_kernel(u, src4d, dst4d, zero_rows)
    u = _layer_mid(aggp, nin, nout, W2, b2.reshape(1, D), zero_c)
    aggp = _agg_kernel(u, src4d, dst4d, zero_rows)
    h = _layer_last(aggp, nin, nout, W3, b3.reshape(1, D), cadd)

    return h.reshape(8, N // 8, D)
